# baseline (speedup 1.0000x reference)
import functools

import jax
import jax.numpy as jnp
from jax import lax
from jax.experimental import pallas as pl
from jax.experimental.pallas import tpu as pltpu
from jax.experimental.pallas import tpu_sc as plsc

T = 32768
D = 128
B = 16
NC = 2
NS = 16
L = 16
NW = NC * NS

T_TC = 27648
T_SC = T - T_TC
R = T_SC // NW
C = 80
NK = R // C
NBUF = 2

BLK = 3456
NBLK = T_TC // BLK

_mesh = plsc.VectorSubcoreMesh(core_axis_name="c", subcore_axis_name="s")


@functools.partial(
    pl.kernel,
    mesh=_mesh,
    out_type=jax.ShapeDtypeStruct((B, NW * L), jnp.float32),
    scratch_types=(
        [pltpu.VMEM((C * D,), jnp.float32) for _ in range(NBUF)]
        + [
            pltpu.VMEM((D,), jnp.float32),
            pltpu.VMEM((L,), jnp.int32),
            pltpu.VMEM((B * L,), jnp.float32),
        ]
        + [pltpu.SemaphoreType.DMA for _ in range(NBUF)]
    ),
)
def _seg_partials(flat_hbm, cu_hbm, w_hbm, out_hbm, *scratch):
    bufs = list(scratch[:NBUF])
    w_v, cu_v, acc_v = scratch[NBUF:NBUF + 3]
    sems = list(scratch[NBUF + 3:])
    wid = lax.axis_index("s") * NC + lax.axis_index("c")
    base = T_TC + wid * R

    copies = [None] * NBUF

    def start(k):
        copies[k % NBUF] = pltpu.async_copy(
            flat_hbm.at[pl.ds((base + k * C) * D, C * D)], bufs[k % NBUF],
            sems[k % NBUF])

    start(0)
    pltpu.sync_copy(w_hbm, w_v)
    pltpu.sync_copy(cu_hbm.at[pl.ds(0, L)], cu_v)

    zero = jnp.zeros((L,), jnp.float32)
    for s in range(B):
        acc_v[pl.ds(s * L, L)] = zero

    wv = [w_v[pl.ds(j * L, L)] for j in range(D // L)]
    cu_vec = cu_v[...]
    cus = [cu_vec[s] for s in range(B)]
    cus.append(jnp.int32(T))
    for k in range(NK):
        if k + 1 < NK:
            start(k + 1)
        copies[k % NBUF].wait()
        buf = bufs[k % NBUF]
        cbase = base + k * C
        for s in range(B):
            lo = jnp.clip(cus[s] - cbase, 0, C)
            hi = jnp.clip(cus[s + 1] - cbase, 0, C)
            acc0 = acc_v[pl.ds(s * L, L)]

            @plsc.parallel_loop(lo * D, hi * D, step=D, unroll=16,
                                carry=acc0)
            def body(rb, a, buf=buf):
                pa = buf[pl.ds(rb, L)] * wv[0]
                for j in range(1, D // L):
                    pa = pa + buf[pl.ds(rb + j * L, L)] * wv[j]
                return a + pa

            acc_v[pl.ds(s * L, L)] = body

    pubs = [pltpu.async_copy(acc_v.at[pl.ds(s * L, L)],
                             out_hbm.at[s, pl.ds(wid * L, L)], sems[0])
            for s in range(B)]
    for cp in pubs:
        cp.wait()


def _cu_columns(cu_ref, lo_off, hi_off):
    sub = lax.broadcasted_iota(jnp.int32, (B, 1), 0)
    lo = jnp.zeros((B, 1), jnp.int32)
    hi = jnp.zeros((B, 1), jnp.int32)
    for s in range(B):
        lo = jnp.where(sub == s, cu_ref[s + lo_off], lo)
        hi = jnp.where(sub == s, cu_ref[s + hi_off], hi)
    return lo, hi


def _tc_partials(flat_ref, cu_ref, w_ref, out_ref):
    i = pl.program_id(0)
    y = jnp.dot(flat_ref[...], w_ref[...],
                preferred_element_type=jnp.float32)
    rows = i * BLK + lax.broadcasted_iota(jnp.int32, (B, BLK), 1)
    lo, hi = _cu_columns(cu_ref, 0, 1)
    onehot_t = ((rows >= lo) & (rows < hi)).astype(jnp.float32)
    partial = jnp.dot(onehot_t, y,
                      preferred_element_type=jnp.float32)

    @pl.when(i == 0)
    def _():
        out_ref[...] = partial

    @pl.when(i > 0)
    def _():
        out_ref[...] += partial


def _epilogue(parts_ref, tc_ref, cu_ref, b_ref, out_ref):
    sums = jnp.sum(parts_ref[...], axis=1, keepdims=True)
    lo, hi = _cu_columns(cu_ref, 0, 1)
    counts = (hi - lo).astype(jnp.float32)
    out_ref[...] = (sums + tc_ref[...]) / counts + b_ref[0]


def kernel(flat, W, b, cu_seqlens):
    flat1d = flat.reshape(T * D)
    w1d = W.reshape(D)

    sc_parts = _seg_partials(flat1d, cu_seqlens, w1d)

    tc_parts = pl.pallas_call(
        _tc_partials,
        grid=(NBLK,),
        in_specs=[
            pl.BlockSpec((BLK, D), lambda i: (i, 0)),
            pl.BlockSpec(memory_space=pltpu.SMEM),
            pl.BlockSpec((D, 1), lambda i: (0, 0)),
        ],
        out_specs=pl.BlockSpec((B, 1), lambda i: (0, 0)),
        out_shape=jax.ShapeDtypeStruct((B, 1), jnp.float32),
    )(flat, cu_seqlens, W)

    out = pl.pallas_call(
        _epilogue,
        in_specs=[
            pl.BlockSpec((B, NW * L), lambda: (0, 0)),
            pl.BlockSpec((B, 1), lambda: (0, 0)),
            pl.BlockSpec(memory_space=pltpu.SMEM),
            pl.BlockSpec(memory_space=pltpu.SMEM),
        ],
        out_shape=jax.ShapeDtypeStruct((B, 1), jnp.float32),
    )(sc_parts, tc_parts, cu_seqlens, b)
    return out


# speedup vs baseline: 1.0045x; 1.0045x over previous
import functools

import jax
import jax.numpy as jnp
from jax import lax
from jax.experimental import pallas as pl
from jax.experimental.pallas import tpu as pltpu
from jax.experimental.pallas import tpu_sc as plsc

T = 32768
D = 128
B = 16
NC = 2
NS = 16
L = 16
NW = NC * NS

T_TC = 26624
T_SC = T - T_TC
R = T_SC // NW
C = 96
NK = R // C
NBUF = 2

BLK = 3328
NBLK = T_TC // BLK

_mesh = plsc.VectorSubcoreMesh(core_axis_name="c", subcore_axis_name="s")


@functools.partial(
    pl.kernel,
    mesh=_mesh,
    out_type=jax.ShapeDtypeStruct((B, NW * L), jnp.float32),
    scratch_types=(
        [pltpu.VMEM((C * D,), jnp.float32) for _ in range(NBUF)]
        + [
            pltpu.VMEM((D,), jnp.float32),
            pltpu.VMEM((L,), jnp.int32),
            pltpu.VMEM((B * L,), jnp.float32),
        ]
        + [pltpu.SemaphoreType.DMA for _ in range(NBUF)]
    ),
)
def _seg_partials(flat_hbm, cu_hbm, w_hbm, out_hbm, *scratch):
    bufs = list(scratch[:NBUF])
    w_v, cu_v, acc_v = scratch[NBUF:NBUF + 3]
    sems = list(scratch[NBUF + 3:])
    wid = lax.axis_index("s") * NC + lax.axis_index("c")
    base = T_TC + wid * R

    copies = [None] * NBUF

    def start(k):
        copies[k % NBUF] = pltpu.async_copy(
            flat_hbm.at[pl.ds((base + k * C) * D, C * D)], bufs[k % NBUF],
            sems[k % NBUF])

    start(0)
    pltpu.sync_copy(w_hbm, w_v)
    pltpu.sync_copy(cu_hbm.at[pl.ds(0, L)], cu_v)

    zero = jnp.zeros((L,), jnp.float32)
    for s in range(B):
        acc_v[pl.ds(s * L, L)] = zero

    wv = [w_v[pl.ds(j * L, L)] for j in range(D // L)]
    cu_vec = cu_v[...]
    cus = [cu_vec[s] for s in range(B)]
    cus.append(jnp.int32(T))
    for k in range(NK):
        if k + 1 < NK:
            start(k + 1)
        copies[k % NBUF].wait()
        buf = bufs[k % NBUF]
        cbase = base + k * C
        for s in range(B):
            lo = jnp.clip(cus[s] - cbase, 0, C)
            hi = jnp.clip(cus[s + 1] - cbase, 0, C)
            acc0 = acc_v[pl.ds(s * L, L)]

            @plsc.parallel_loop(lo * D, hi * D, step=D, unroll=16,
                                carry=acc0)
            def body(rb, a, buf=buf):
                pa = buf[pl.ds(rb, L)] * wv[0]
                for j in range(1, D // L):
                    pa = pa + buf[pl.ds(rb + j * L, L)] * wv[j]
                return a + pa

            acc_v[pl.ds(s * L, L)] = body

    pubs = [pltpu.async_copy(acc_v.at[pl.ds(s * L, L)],
                             out_hbm.at[s, pl.ds(wid * L, L)], sems[0])
            for s in range(B)]
    for cp in pubs:
        cp.wait()


def _cu_columns(cu_ref, lo_off, hi_off):
    sub = lax.broadcasted_iota(jnp.int32, (B, 1), 0)
    lo = jnp.zeros((B, 1), jnp.int32)
    hi = jnp.zeros((B, 1), jnp.int32)
    for s in range(B):
        lo = jnp.where(sub == s, cu_ref[s + lo_off], lo)
        hi = jnp.where(sub == s, cu_ref[s + hi_off], hi)
    return lo, hi


def _tc_partials(flat_ref, cu_ref, w_ref, out_ref):
    i = pl.program_id(0)
    y = jnp.dot(flat_ref[...], w_ref[...],
                preferred_element_type=jnp.float32)
    rows = i * BLK + lax.broadcasted_iota(jnp.int32, (B, BLK), 1)
    lo, hi = _cu_columns(cu_ref, 0, 1)
    onehot_t = ((rows >= lo) & (rows < hi)).astype(jnp.float32)
    partial = jnp.dot(onehot_t, y,
                      preferred_element_type=jnp.float32)

    @pl.when(i == 0)
    def _():
        out_ref[...] = partial

    @pl.when(i > 0)
    def _():
        out_ref[...] += partial


def _epilogue(parts_ref, tc_ref, cu_ref, b_ref, out_ref):
    sums = jnp.sum(parts_ref[...], axis=1, keepdims=True)
    lo, hi = _cu_columns(cu_ref, 0, 1)
    counts = (hi - lo).astype(jnp.float32)
    out_ref[...] = (sums + tc_ref[...]) / counts + b_ref[0]


def kernel(flat, W, b, cu_seqlens):
    flat1d = flat.reshape(T * D)
    w1d = W.reshape(D)

    sc_parts = _seg_partials(flat1d, cu_seqlens, w1d)

    tc_parts = pl.pallas_call(
        _tc_partials,
        grid=(NBLK,),
        in_specs=[
            pl.BlockSpec((BLK, D), lambda i: (i, 0)),
            pl.BlockSpec(memory_space=pltpu.SMEM),
            pl.BlockSpec((D, 1), lambda i: (0, 0)),
        ],
        out_specs=pl.BlockSpec((B, 1), lambda i: (0, 0)),
        out_shape=jax.ShapeDtypeStruct((B, 1), jnp.float32),
    )(flat, cu_seqlens, W)

    out = pl.pallas_call(
        _epilogue,
        in_specs=[
            pl.BlockSpec((B, NW * L), lambda: (0, 0)),
            pl.BlockSpec((B, 1), lambda: (0, 0)),
            pl.BlockSpec(memory_space=pltpu.SMEM),
            pl.BlockSpec(memory_space=pltpu.SMEM),
        ],
        out_shape=jax.ShapeDtypeStruct((B, 1), jnp.float32),
    )(sc_parts, tc_parts, cu_seqlens, b)
    return out
